# baseline (speedup 1.0000x reference)
import functools

import jax
import jax.numpy as jnp
import numpy as np
from jax import lax
from jax.experimental import pallas as pl
from jax.experimental.pallas import tpu as pltpu
from jax.experimental.pallas import tpu_sc as plsc

_Z_BINS = 64
_MAX_Z = 127
_EMBED_DIM = 128
_BATCH = 16384

_NUM_WORKERS = 32
_ROWS_PER_WORKER = _BATCH // _NUM_WORKERS
_IDX_CHUNK = 128
_CHUNKS = _ROWS_PER_WORKER // _IDX_CHUNK
_TABLE_ROWS = 136


def _sin_rows() -> np.ndarray:
    position = np.arange(_MAX_Z)[:, None].astype(np.float32)
    div_term = np.exp(
        np.arange(0, _EMBED_DIM, 2).astype(np.float32)
        * (-np.log(10000.0) / _EMBED_DIM)
    )
    pe = np.zeros((_MAX_Z, _EMBED_DIM), dtype=np.float32)
    pe[:, 0::2] = np.sin(position * div_term)
    pe[:, 1::2] = np.cos(position * div_term)
    z_bin = np.arange(_Z_BINS, dtype=np.float32)
    z_idx = ((z_bin + 0.5) / _Z_BINS * _MAX_Z).astype(np.int32)
    z_idx = np.clip(z_idx, 0, _MAX_Z - 1)
    return pe[z_idx]


_SIN = _sin_rows()


def _table_body(sin_ref, path_ref, bin_ref, wzc_ref, bzc_ref, wc_ref, bc_ref,
                null_ref, out_ref):
    def dot_t(a, b):
        return lax.dot_general(a, b, (((1,), (1,)), ((), ())),
                               preferred_element_type=jnp.float32)

    z_emb = (dot_t(bin_ref[:], wzc_ref[:, 0:_EMBED_DIM])
             + dot_t(sin_ref[:], wzc_ref[:, _EMBED_DIM:2 * _EMBED_DIM])
             + bzc_ref[:])
    z_part = dot_t(z_emb, wc_ref[:, _EMBED_DIM:2 * _EMBED_DIM]) + bc_ref[:]
    p_part = dot_t(path_ref[:], wc_ref[:, 0:_EMBED_DIM])
    out_ref[0:_Z_BINS, :] = z_part + jnp.broadcast_to(
        p_part[0:1, :], (_Z_BINS, _EMBED_DIM))
    out_ref[_Z_BINS:_EMBED_DIM, :] = z_part + jnp.broadcast_to(
        p_part[1:2, :], (_Z_BINS, _EMBED_DIM))
    out_ref[_EMBED_DIM:_TABLE_ROWS, :] = jnp.broadcast_to(
        null_ref[:], (_TABLE_ROWS - _EMBED_DIM, _EMBED_DIM))


def _build_table(path_table, bin_table, W_zc, b_zc, W_c, b_c, null_emb):
    return pl.pallas_call(
        _table_body,
        out_shape=jax.ShapeDtypeStruct((_TABLE_ROWS, _EMBED_DIM), jnp.float32),
    )(_SIN, path_table, bin_table, W_zc, b_zc.reshape(1, _EMBED_DIM),
      W_c, b_c.reshape(1, _EMBED_DIM), null_emb)


_TABLE_WORDS = _TABLE_ROWS * _EMBED_DIM
_CHUNK_WORDS = _IDX_CHUNK * _EMBED_DIM
_LANES = 16
_GROUPS = _EMBED_DIM // _LANES


@functools.cache
def _make_gather():
    @functools.partial(
        pl.kernel,
        mesh=plsc.VectorSubcoreMesh(core_axis_name="c", subcore_axis_name="s"),
        out_type=jax.ShapeDtypeStruct(
            (_NUM_WORKERS, _CHUNKS, _IDX_CHUNK, _EMBED_DIM), jnp.float32),
        scratch_types=[
            pltpu.VMEM_SHARED((_TABLE_ROWS, _EMBED_DIM), jnp.float32),
            pltpu.VMEM((_CHUNKS, _IDX_CHUNK), jnp.int32),
            pltpu.VMEM((_CHUNKS, _IDX_CHUNK, _EMBED_DIM), jnp.float32),
            pltpu.SemaphoreType.DMA,
            pltpu.SemaphoreType.DMA,
        ],
    )
    def _gather_rows(table_hbm, tokens_hbm, out_hbm, table_sh, idx_v, rows_v,
                     gsem, osem):
        wid = lax.axis_index("s") * 2 + lax.axis_index("c")
        @pl.when(lax.axis_index("s") == 0)
        def _():
            pltpu.sync_copy(table_hbm, table_sh)

        pltpu.sync_copy(tokens_hbm.at[wid], idx_v)
        plsc.subcore_barrier()
        gathers = [
            pltpu.async_copy(table_sh.at[idx_v.at[j]], rows_v.at[j], gsem)
            for j in range(_CHUNKS)
        ]
        outs = []
        for j in range(_CHUNKS):
            gathers[j].wait()
            outs.append(
                pltpu.async_copy(rows_v.at[j], out_hbm.at[wid, j], osem))
        for o in outs:
            o.wait()

    return _gather_rows


def kernel(tokens, path_table, bin_table, W_zc, b_zc, W_c, b_c, null_emb):
    table = _build_table(path_table, bin_table, W_zc, b_zc, W_c, b_c, null_emb)
    tokens3 = tokens.reshape(_NUM_WORKERS, _CHUNKS, _IDX_CHUNK)
    out = _make_gather()(table, tokens3)
    return out.reshape(_BATCH, _EMBED_DIM)
